# baseline (speedup 1.0000x reference)
import functools

import jax
import jax.numpy as jnp
from jax import lax
from jax.experimental import pallas as pl
from jax.experimental.pallas import tpu as pltpu
from jax.experimental.pallas import tpu_sc as plsc

_CHUNK = 8192
_NMOM = 50
_CSTRIDE = 16


def _sc_moments(n2g, n2g_coarse, cols, num_graphs):
    N = n2g.shape[0]
    assert N >= _CHUNK and (N - _CHUNK) % 8 == 0
    NCRS = n2g_coarse.shape[0]
    info = plsc.get_sparse_core_info()
    NC, NS, L = info.num_cores, info.num_subcores, info.num_lanes
    NW = NC * NS
    assert NW * L == num_graphs
    cbits = max(1, (NCRS - 1).bit_length())
    mesh = plsc.VectorSubcoreMesh(core_axis_name="c", subcore_axis_name="s")

    @functools.partial(
        pl.kernel,
        mesh=mesh,
        compiler_params=pltpu.CompilerParams(needs_layout_passes=False),
        out_type=jax.ShapeDtypeStruct((NW * _NMOM * L,), jnp.float32),
        scratch_types=[pltpu.VMEM((NCRS,), jnp.int32)]
        + [pltpu.VMEM((3 * _CHUNK,), jnp.float32)] * 3
        + [pltpu.VMEM((_NMOM * L,), jnp.float32)]
        + [pltpu.VMEM((8 * L,), jnp.int32)] * 2
        + [pltpu.VMEM((_CSTRIDE * L,), jnp.int32)]
        + [pltpu.SemaphoreType.DMA],
    )
    def body(n2g_hbm, crs_hbm, pth, pph, pnh, out_hbm, crsv, bpt, bpp, bpn,
             outv, idx0, idx1, finev, sem):
        colsh = (pth, pph, pnh)
        bufs = (bpt, bpp, bpn)
        wid = lax.axis_index("s") * NC + lax.axis_index("c")
        pltpu.sync_copy(crs_hbm, crsv)
        lane = lax.broadcasted_iota(jnp.int32, (L,), 0)
        g = wid * L + lane

        def lower_bound(target):
            def itc(_, lh):
                lo, hi = lh
                mid = jnp.minimum((lo + hi) >> 1, NCRS - 1)
                v = plsc.load_gather(crsv, [mid])
                pred = v < target
                return jnp.where(pred, mid + 1, lo), jnp.where(pred, hi, mid)

            jc, _ = lax.fori_loop(
                0, cbits, itc,
                (jnp.zeros((L,), jnp.int32), jnp.full((L,), NCRS, jnp.int32)),
            )
            w = jnp.maximum(_CSTRIDE * jc - (_CSTRIDE - 1), 0)
            for k in range(_CSTRIDE):
                ent = jnp.clip(w + k, 0, N - 1)
                if k < 8:
                    idx0[pl.ds(k * L, L)] = ent
                else:
                    idx1[pl.ds((k - 8) * L, L)] = ent
            cp0 = pltpu.async_copy(n2g_hbm.at[idx0], finev.at[pl.ds(0, 8 * L)], sem)
            cp1 = pltpu.async_copy(n2g_hbm.at[idx1], finev.at[pl.ds(8 * L, 8 * L)], sem)
            cp0.wait()
            cp1.wait()

            def itf(_, lh):
                lo, hi = lh
                mid = jnp.minimum((lo + hi) >> 1, _CSTRIDE - 1)
                v = plsc.load_gather(finev, [mid * L + lane])
                pred = v < target
                return jnp.where(pred, mid + 1, lo), jnp.where(pred, hi, mid)

            flo, _ = lax.fori_loop(
                0, 5, itf,
                (jnp.zeros((L,), jnp.int32), jnp.full((L,), _CSTRIDE, jnp.int32)),
            )
            return jnp.minimum(w + flo, N)

        s = lower_bound(g)
        e = lower_bound(g + 1)
        c0 = (jnp.min(s) >> 3) << 3
        hi_t = jnp.max(e)
        accs0 = [jnp.zeros((L,), jnp.float32) for _ in range(_NMOM - 1)]

        def chunk_cond(st):
            return st[0] < hi_t

        def chunk_body(st):
            c, ptr = st[0], st[1]
            accs = st[2:]
            ca = jnp.minimum(c, N - _CHUNK)
            ca3 = pl.multiple_of(ca * 3, 8)
            cps = [
                pltpu.async_copy(colsh[k].at[pl.ds(ca3, 3 * _CHUNK)], bufs[k], sem)
                for k in range(3)
            ]
            for cp in cps:
                cp.wait()
            cend = c + _CHUNK
            rem = jnp.maximum(jnp.minimum(e, cend) - ptr, 0)
            T = jnp.max(rem)

            def it(_, st2):
                p, *a = st2
                active = (p < e) & (p < cend)
                base = jnp.clip(p - ca, 0, _CHUNK - 1) * 3
                fm = jnp.where(active, 1.0, 0.0).astype(jnp.float32)
                v = [plsc.load_gather(bufs[k], [base + j]) * fm
                     for k in range(3) for j in range(3)]
                ptx, pty, ptz, ppx, ppy, ppz, pnx, pny, pnz = v
                prods = [
                    ptx, pty, ptz,
                    ppx, ppy, ppz,
                    ppx * ppx, ppx * ppy, ppx * ppz,
                    ppy * ppy, ppy * ppz, ppz * ppz,
                    ptx * ptx, ptx * pty, ptx * ptz,
                    pty * pty, pty * ptz, ptz * ptz,
                    ptx * ppx, ptx * ppy, ptx * ppz,
                    pty * ppx, pty * ppy, pty * ppz,
                    ptz * ppx, ptz * ppy, ptz * ppz,
                    pnx, pny, pnz,
                    ppx * pnx, ppx * pny, ppx * pnz,
                    ppy * pnx, ppy * pny, ppy * pnz,
                    ppz * pnx, ppz * pny, ppz * pnz,
                    ptx * pnx, ptx * pny, ptx * pnz,
                    pty * pnx, pty * pny, pty * pnz,
                    ptz * pnx, ptz * pny, ptz * pnz,
                    pnx * pnx + pny * pny + pnz * pnz,
                ]
                p = p + jnp.where(active, 1, 0)
                return (p, *[x + q for x, q in zip(a, prods)])

            st2 = lax.fori_loop(0, T, it, (ptr, *accs))
            return (c + _CHUNK, *st2)

        st = lax.while_loop(chunk_cond, chunk_body, (c0, s, *accs0))
        accs = st[2:]
        cnt = (e - s).astype(jnp.float32)
        vals = [cnt] + list(accs)
        for m in range(_NMOM):
            outv[pl.ds(m * L, L)] = vals[m]
        pltpu.sync_copy(outv, out_hbm.at[pl.ds(wid * _NMOM * L, _NMOM * L)])

    return body(n2g, n2g_coarse, *cols)


def _tc_finalize(M, mr, W1T, b1r, W2T, b2r, nl_row, nl_col, sig_col, num_graphs):
    G = num_graphs
    EMB = mr.shape[1]
    NL = sig_col.shape[0]

    def body(m_ref, mr_ref, w1_ref, b1_ref, w2_ref, b2_ref, nlr_ref, nlc_ref,
             sig_ref, o1_ref, o2_ref):
        row = lambda i: m_ref[i:i + 1, :]
        cnt = row(0)
        Spt = [row(1 + i) for i in range(3)]
        Spp = [row(4 + i) for i in range(3)]

        def sym(b):
            r = [row(b), row(b + 1), row(b + 2), row(b + 3), row(b + 4), row(b + 5)]
            return [[r[0], r[1], r[2]], [r[1], r[3], r[4]], [r[2], r[4], r[5]]]

        def full(b):
            return [[row(b + 3 * i + j) for j in range(3)] for i in range(3)]

        Spp_pp = sym(7)
        Spt_pt = sym(13)
        Spt_pp = full(19)
        Spn = [row(28 + i) for i in range(3)]
        Spp_pn = full(31)
        Spt_pn = full(40)
        Snn = row(49)
        Sd = [Spn[i] - Spp[i] for i in range(3)]
        Spp_d = [[Spp_pn[i][j] - Spp_pp[i][j] for j in range(3)] for i in range(3)]
        Spt_d = [[Spt_pn[i][j] - Spt_pp[i][j] for j in range(3)] for i in range(3)]
        Sdd = (Snn - 2.0 * (Spp_pn[0][0] + Spp_pn[1][1] + Spp_pn[2][2])
               + Spp_pp[0][0] + Spp_pp[1][1] + Spp_pp[2][2])

        r1 = 1.0 / jnp.maximum(cnt, 1.0)
        P = [[Spp_pp[i][j] - Spp[i] * Spp[j] * r1 for j in range(3)] for i in range(3)]
        O = [[Spt_pp[i][j] - Spt[i] * Spp[j] * r1 for j in range(3)] for i in range(3)]
        Tm = [[Spt_pt[i][j] - Spt[i] * Spt[j] * r1 for j in range(3)] for i in range(3)]
        Cpd = [[Spp_d[i][j] - Spp[i] * Sd[j] * r1 for j in range(3)] for i in range(3)]
        Ctd = [[Spt_d[i][j] - Spt[i] * Sd[j] * r1 for j in range(3)] for i in range(3)]

        def fr2(A):
            t = A[0][0] * 0.0
            for i in range(3):
                for j in range(3):
                    t = t + A[i][j] * A[i][j]
            return t

        den = jnp.sqrt(fr2(P)) + jnp.sqrt(fr2(O))
        X = P[0][0] * 0.0
        for i in range(3):
            for j in range(3):
                X = X + P[i][j] * Cpd[i][j] - O[i][j] * Ctd[i][j]
        Y = P[0][0] * 0.0
        for i in range(3):
            for j in range(3):
                for k in range(3):
                    Y = Y + (P[i][j] * P[j][k] * P[k][i]
                             - 2.0 * P[i][j] * O[k][j] * O[k][i]
                             + O[i][j] * O[k][j] * Tm[i][k])

        nlr = nlr_ref[0:1, :]
        lev = lax.broadcasted_iota(jnp.int32, (NL, G), 0)
        oh = jnp.where(lev == nlr, 1.0, 0.0)
        sig = jnp.sum(sig_ref[:, :] * oh, axis=0, keepdims=True)
        inv_s2 = 1.0 / (sig * sig)
        A2 = (Sdd + (4.0 / den) * X + (4.0 / (den * den)) * Y) * inv_s2
        A2 = jnp.where(cnt > 0.0, A2, 0.0)
        o1_ref[:, :] = jnp.sum(A2, axis=1, keepdims=True) / G

        x = mr_ref[:, :]
        h = jnp.dot(x, w1_ref[:, :], preferred_element_type=jnp.float32) + b1_ref[0:1, :]
        h = h / (1.0 + jnp.exp(-h))
        p = jnp.dot(h, w2_ref[:, :], preferred_element_type=jnp.float32) + b2_ref[0:1, :]
        mx = jnp.max(p, axis=1, keepdims=True)
        lse = jnp.log(jnp.sum(jnp.exp(p - mx), axis=1, keepdims=True)) + mx
        cls = lax.broadcasted_iota(jnp.int32, (G, EMB), 1)
        sel = jnp.sum(jnp.where(cls == nlc_ref[:, :], p, 0.0), axis=1, keepdims=True)
        o2_ref[:, :] = jnp.sum(lse - sel, axis=0, keepdims=True) / G

    o1, o2 = pl.pallas_call(
        body,
        out_shape=[
            jax.ShapeDtypeStruct((1, 1), jnp.float32),
            jax.ShapeDtypeStruct((1, 1), jnp.float32),
        ],
    )(M, mr, W1T, b1r, W2T, b2r, nl_row, nl_col, sig_col)
    return o1, o2


def kernel(node2graph, edge_index, num_graphs, energy, molecule_repr,
           pos_noise_pred, pos_perturbed, pos_target, sigmas, noise_level,
           W1, b1, W2, b2):
    del edge_index, energy, num_graphs
    G = molecule_repr.shape[0]
    n2g = node2graph.astype(jnp.int32)

    cols = [pos_target.reshape(-1), pos_perturbed.reshape(-1),
            pos_noise_pred.reshape(-1)]

    crs = n2g[::_CSTRIDE]
    pad_c = (-crs.shape[0]) % 8
    crs = jnp.pad(crs, (0, pad_c), constant_values=jnp.int32(1 << 30))

    flat = _sc_moments(n2g, crs, cols, G)
    info = plsc.get_sparse_core_info()
    NW = info.num_cores * info.num_subcores
    L = info.num_lanes
    M = jnp.transpose(flat.reshape(NW, _NMOM, L), (1, 0, 2)).reshape(_NMOM, NW * L)

    nl = noise_level.astype(jnp.int32)
    o1, o2 = _tc_finalize(
        M,
        molecule_repr,
        W1.T, b1.reshape(1, -1),
        W2.T, b2.reshape(1, -1),
        nl.reshape(1, G), nl.reshape(G, 1),
        sigmas.reshape(-1, 1),
        G,
    )
    return (o1.reshape(()), o2.reshape(()))


# speedup vs baseline: 3.7048x; 3.7048x over previous
import functools

import jax
import jax.numpy as jnp
from jax import lax
from jax.experimental import pallas as pl
from jax.experimental.pallas import tpu as pltpu
from jax.experimental.pallas import tpu_sc as plsc

_CHUNK = 8192
_NMOM = 50
_CSTRIDE = 16


def _sc_moments(n2g, n2g_coarse, cols, num_graphs):
    N = n2g.shape[0]
    assert N >= _CHUNK and (N - _CHUNK) % 8 == 0
    NCRS = n2g_coarse.shape[0]
    info = plsc.get_sparse_core_info()
    NC, NS, L = info.num_cores, info.num_subcores, info.num_lanes
    NW = NC * NS
    assert NW * L == num_graphs
    cbits = max(1, (NCRS - 1).bit_length())
    mesh = plsc.VectorSubcoreMesh(core_axis_name="c", subcore_axis_name="s")

    @functools.partial(
        pl.kernel,
        mesh=mesh,
        compiler_params=pltpu.CompilerParams(needs_layout_passes=False),
        out_type=jax.ShapeDtypeStruct((NW * _NMOM * L,), jnp.float32),
        scratch_types=[pltpu.VMEM((NCRS,), jnp.int32)]
        + [pltpu.VMEM((_CHUNK,), jnp.float32)] * 9
        + [pltpu.VMEM((_NMOM * L,), jnp.float32)]
        + [pltpu.VMEM((8 * L,), jnp.int32)] * 2
        + [pltpu.VMEM((_CSTRIDE * L,), jnp.int32)]
        + [pltpu.SemaphoreType.DMA],
    )
    def body(n2g_hbm, crs_hbm, c0h, c1h, c2h, c3h, c4h, c5h, c6h, c7h, c8h,
             out_hbm, crsv, b0, b1, b2, b3, b4, b5, b6, b7, b8,
             outv, idx0, idx1, finev, sem):
        colsh = (c0h, c1h, c2h, c3h, c4h, c5h, c6h, c7h, c8h)
        bufs = (b0, b1, b2, b3, b4, b5, b6, b7, b8)
        wid = lax.axis_index("s") * NC + lax.axis_index("c")
        pltpu.sync_copy(crs_hbm, crsv)
        lane = lax.broadcasted_iota(jnp.int32, (L,), 0)
        g = wid * L + lane

        def lower_bound(target):
            def itc(_, lh):
                lo, hi = lh
                mid = jnp.minimum((lo + hi) >> 1, NCRS - 1)
                v = plsc.load_gather(crsv, [mid])
                pred = v < target
                return jnp.where(pred, mid + 1, lo), jnp.where(pred, hi, mid)

            jc, _ = lax.fori_loop(
                0, cbits, itc,
                (jnp.zeros((L,), jnp.int32), jnp.full((L,), NCRS, jnp.int32)),
            )
            w = jnp.maximum(_CSTRIDE * jc - (_CSTRIDE - 1), 0)
            for k in range(_CSTRIDE):
                ent = jnp.clip(w + k, 0, N - 1)
                if k < 8:
                    idx0[pl.ds(k * L, L)] = ent
                else:
                    idx1[pl.ds((k - 8) * L, L)] = ent
            cp0 = pltpu.async_copy(n2g_hbm.at[idx0], finev.at[pl.ds(0, 8 * L)], sem)
            cp1 = pltpu.async_copy(n2g_hbm.at[idx1], finev.at[pl.ds(8 * L, 8 * L)], sem)
            cp0.wait()
            cp1.wait()

            def itf(_, lh):
                lo, hi = lh
                mid = jnp.minimum((lo + hi) >> 1, _CSTRIDE - 1)
                v = plsc.load_gather(finev, [mid * L + lane])
                pred = v < target
                return jnp.where(pred, mid + 1, lo), jnp.where(pred, hi, mid)

            flo, _ = lax.fori_loop(
                0, 5, itf,
                (jnp.zeros((L,), jnp.int32), jnp.full((L,), _CSTRIDE, jnp.int32)),
            )
            return jnp.minimum(w + flo, N)

        s = lower_bound(g)
        e = lower_bound(g + 1)
        c0 = (jnp.min(s) >> 3) << 3
        hi_t = jnp.max(e)
        accs0 = [jnp.zeros((L,), jnp.float32) for _ in range(_NMOM - 1)]

        def chunk_cond(st):
            return st[0] < hi_t

        def chunk_body(st):
            c, ptr = st[0], st[1]
            accs = st[2:]
            ca = pl.multiple_of(jnp.minimum(c, N - _CHUNK), 8)
            cps = [
                pltpu.async_copy(colsh[k].at[pl.ds(ca, _CHUNK)], bufs[k], sem)
                for k in range(9)
            ]
            for cp in cps:
                cp.wait()
            cend = c + _CHUNK
            rem = jnp.maximum(jnp.minimum(e, cend) - ptr, 0)
            T = jnp.max(rem)

            def it(_, st2):
                p, *a = st2
                active = (p < e) & (p < cend)
                idxl = jnp.clip(p - ca, 0, _CHUNK - 1)
                fm = jnp.where(active, 1.0, 0.0).astype(jnp.float32)
                v = [plsc.load_gather(bufs[k], [idxl]) * fm for k in range(9)]
                ptx, pty, ptz, ppx, ppy, ppz, pnx, pny, pnz = v
                prods = [
                    ptx, pty, ptz,
                    ppx, ppy, ppz,
                    ppx * ppx, ppx * ppy, ppx * ppz,
                    ppy * ppy, ppy * ppz, ppz * ppz,
                    ptx * ptx, ptx * pty, ptx * ptz,
                    pty * pty, pty * ptz, ptz * ptz,
                    ptx * ppx, ptx * ppy, ptx * ppz,
                    pty * ppx, pty * ppy, pty * ppz,
                    ptz * ppx, ptz * ppy, ptz * ppz,
                    pnx, pny, pnz,
                    ppx * pnx, ppx * pny, ppx * pnz,
                    ppy * pnx, ppy * pny, ppy * pnz,
                    ppz * pnx, ppz * pny, ppz * pnz,
                    ptx * pnx, ptx * pny, ptx * pnz,
                    pty * pnx, pty * pny, pty * pnz,
                    ptz * pnx, ptz * pny, ptz * pnz,
                    pnx * pnx + pny * pny + pnz * pnz,
                ]
                p = p + jnp.where(active, 1, 0)
                return (p, *[x + q for x, q in zip(a, prods)])

            st2 = lax.fori_loop(0, T, it, (ptr, *accs))
            return (c + _CHUNK, *st2)

        st = lax.while_loop(chunk_cond, chunk_body, (c0, s, *accs0))
        accs = st[2:]
        cnt = (e - s).astype(jnp.float32)
        vals = [cnt] + list(accs)
        for m in range(_NMOM):
            outv[pl.ds(m * L, L)] = vals[m]
        pltpu.sync_copy(outv, out_hbm.at[pl.ds(wid * _NMOM * L, _NMOM * L)])

    return body(n2g, n2g_coarse, *cols)


def _tc_finalize(M, mr, W1T, b1r, W2T, b2r, nl_row, nl_col, sig_col, num_graphs):
    G = num_graphs
    EMB = mr.shape[1]
    NL = sig_col.shape[0]

    def body(m_ref, mr_ref, w1_ref, b1_ref, w2_ref, b2_ref, nlr_ref, nlc_ref,
             sig_ref, o1_ref, o2_ref):
        row = lambda i: m_ref[i:i + 1, :]
        cnt = row(0)
        Spt = [row(1 + i) for i in range(3)]
        Spp = [row(4 + i) for i in range(3)]

        def sym(b):
            r = [row(b), row(b + 1), row(b + 2), row(b + 3), row(b + 4), row(b + 5)]
            return [[r[0], r[1], r[2]], [r[1], r[3], r[4]], [r[2], r[4], r[5]]]

        def full(b):
            return [[row(b + 3 * i + j) for j in range(3)] for i in range(3)]

        Spp_pp = sym(7)
        Spt_pt = sym(13)
        Spt_pp = full(19)
        Spn = [row(28 + i) for i in range(3)]
        Spp_pn = full(31)
        Spt_pn = full(40)
        Snn = row(49)
        Sd = [Spn[i] - Spp[i] for i in range(3)]
        Spp_d = [[Spp_pn[i][j] - Spp_pp[i][j] for j in range(3)] for i in range(3)]
        Spt_d = [[Spt_pn[i][j] - Spt_pp[i][j] for j in range(3)] for i in range(3)]
        Sdd = (Snn - 2.0 * (Spp_pn[0][0] + Spp_pn[1][1] + Spp_pn[2][2])
               + Spp_pp[0][0] + Spp_pp[1][1] + Spp_pp[2][2])

        r1 = 1.0 / jnp.maximum(cnt, 1.0)
        P = [[Spp_pp[i][j] - Spp[i] * Spp[j] * r1 for j in range(3)] for i in range(3)]
        O = [[Spt_pp[i][j] - Spt[i] * Spp[j] * r1 for j in range(3)] for i in range(3)]
        Tm = [[Spt_pt[i][j] - Spt[i] * Spt[j] * r1 for j in range(3)] for i in range(3)]
        Cpd = [[Spp_d[i][j] - Spp[i] * Sd[j] * r1 for j in range(3)] for i in range(3)]
        Ctd = [[Spt_d[i][j] - Spt[i] * Sd[j] * r1 for j in range(3)] for i in range(3)]

        def fr2(A):
            t = A[0][0] * 0.0
            for i in range(3):
                for j in range(3):
                    t = t + A[i][j] * A[i][j]
            return t

        den = jnp.sqrt(fr2(P)) + jnp.sqrt(fr2(O))
        X = P[0][0] * 0.0
        for i in range(3):
            for j in range(3):
                X = X + P[i][j] * Cpd[i][j] - O[i][j] * Ctd[i][j]
        Y = P[0][0] * 0.0
        for i in range(3):
            for j in range(3):
                for k in range(3):
                    Y = Y + (P[i][j] * P[j][k] * P[k][i]
                             - 2.0 * P[i][j] * O[k][j] * O[k][i]
                             + O[i][j] * O[k][j] * Tm[i][k])

        nlr = nlr_ref[0:1, :]
        lev = lax.broadcasted_iota(jnp.int32, (NL, G), 0)
        oh = jnp.where(lev == nlr, 1.0, 0.0)
        sig = jnp.sum(sig_ref[:, :] * oh, axis=0, keepdims=True)
        inv_s2 = 1.0 / (sig * sig)
        A2 = (Sdd + (4.0 / den) * X + (4.0 / (den * den)) * Y) * inv_s2
        A2 = jnp.where(cnt > 0.0, A2, 0.0)
        o1_ref[:, :] = jnp.sum(A2, axis=1, keepdims=True) / G

        x = mr_ref[:, :]
        h = jnp.dot(x, w1_ref[:, :], preferred_element_type=jnp.float32) + b1_ref[0:1, :]
        h = h / (1.0 + jnp.exp(-h))
        p = jnp.dot(h, w2_ref[:, :], preferred_element_type=jnp.float32) + b2_ref[0:1, :]
        mx = jnp.max(p, axis=1, keepdims=True)
        lse = jnp.log(jnp.sum(jnp.exp(p - mx), axis=1, keepdims=True)) + mx
        cls = lax.broadcasted_iota(jnp.int32, (G, EMB), 1)
        sel = jnp.sum(jnp.where(cls == nlc_ref[:, :], p, 0.0), axis=1, keepdims=True)
        o2_ref[:, :] = jnp.sum(lse - sel, axis=0, keepdims=True) / G

    o1, o2 = pl.pallas_call(
        body,
        out_shape=[
            jax.ShapeDtypeStruct((1, 1), jnp.float32),
            jax.ShapeDtypeStruct((1, 1), jnp.float32),
        ],
    )(M, mr, W1T, b1r, W2T, b2r, nl_row, nl_col, sig_col)
    return o1, o2


def kernel(node2graph, edge_index, num_graphs, energy, molecule_repr,
           pos_noise_pred, pos_perturbed, pos_target, sigmas, noise_level,
           W1, b1, W2, b2):
    del edge_index, energy, num_graphs
    G = molecule_repr.shape[0]
    n2g = node2graph.astype(jnp.int32)

    cols = []
    for arr in (pos_target, pos_perturbed, pos_noise_pred):
        for i in range(3):
            cols.append(arr[:, i])

    crs = n2g[::_CSTRIDE]
    pad_c = (-crs.shape[0]) % 8
    crs = jnp.pad(crs, (0, pad_c), constant_values=jnp.int32(1 << 30))

    flat = _sc_moments(n2g, crs, cols, G)
    info = plsc.get_sparse_core_info()
    NW = info.num_cores * info.num_subcores
    L = info.num_lanes
    M = jnp.transpose(flat.reshape(NW, _NMOM, L), (1, 0, 2)).reshape(_NMOM, NW * L)

    nl = noise_level.astype(jnp.int32)
    o1, o2 = _tc_finalize(
        M,
        molecule_repr,
        W1.T, b1.reshape(1, -1),
        W2.T, b2.reshape(1, -1),
        nl.reshape(1, G), nl.reshape(G, 1),
        sigmas.reshape(-1, 1),
        G,
    )
    return (o1.reshape(()), o2.reshape(()))
